# baseline (speedup 1.0000x reference)
import functools

import jax
import jax.numpy as jnp
from jax.experimental import pallas as pl
from jax.experimental.pallas import tpu as pltpu

_F32 = jnp.float32
_BF16 = jnp.bfloat16


def _qkv_kernel(x_ref, wk_ref, wq_ref, wv_ref, bk_ref, bq_ref, bv_ref,
                k_ref, q_ref, v_ref):
    xb = x_ref[0]
    k = jnp.dot(wk_ref[...], xb, preferred_element_type=_F32) + bk_ref[...]
    k_ref[0] = k.astype(_BF16)
    q = jnp.dot(wq_ref[...], xb, preferred_element_type=_F32) + bq_ref[...]
    q_ref[0] = q.astype(_BF16)
    v = jnp.dot(wv_ref[...], xb, preferred_element_type=_F32) + bv_ref[...]
    v_ref[0] = v.astype(_BF16)


def _attn_kernel(nbatch, nd, bd, k_ref, q_hbm, v_ref, o_ref,
                 s_ref, acc_ref, q_vmem, q_sem):
    c_idx = pl.program_id(0)
    d_idx = pl.program_id(1)

    @pl.when((c_idx == 0) & (d_idx == 0))
    def _():
        pltpu.make_async_copy(q_hbm, q_vmem, q_sem).start()
        pltpu.make_async_copy(q_hbm, q_vmem, q_sem).wait()

    @pl.when(d_idx == 0)
    def _():
        acc_ref[...] = jnp.zeros_like(acc_ref)

    m = None
    for b in range(nbatch):
        s = jax.lax.dot_general(
            k_ref[b], q_vmem[b, pl.ds(d_idx * bd, bd), :],
            dimension_numbers=(((1,), (1,)), ((), ())),
            preferred_element_type=_F32)
        s_ref[b] = s
        m = s if m is None else jnp.maximum(m, s)

    norm = jnp.zeros_like(m)
    for b in range(nbatch):
        e = jnp.exp(s_ref[b] - m)
        s_ref[b] = e
        norm = norm + e
    inv = 1.0 / norm

    for b in range(nbatch):
        attn = (s_ref[b] * inv).astype(_BF16)
        pv = jnp.dot(attn, v_ref[b], preferred_element_type=_F32)
        acc_ref[b] = (acc_ref[b].astype(_F32) + pv).astype(_BF16)

    @pl.when(d_idx == nd - 1)
    def _():
        o_ref[...] = acc_ref[...]


def _proj_kernel(alpha_ref, att_ref, x_ref, wr_ref, br_ref, o_ref):
    a = alpha_ref[0, 0]
    att = jnp.dot(wr_ref[...], att_ref[0].astype(_F32),
                  preferred_element_type=_F32) + br_ref[...]
    o_ref[0] = a * att + x_ref[0]


def _pick_tile(c, target, quantum):
    t = min(target, c)
    while t > quantum and c % t:
        t -= quantum
    return t if c % t == 0 else c


@functools.partial(jax.jit, static_argnames=())
def kernel(x, Wq, bq, Wk, bk, Wv, bv, Wr, br, alpha):
    B, C, S0, S1 = x.shape
    HW = S0 * S1
    x3 = x.reshape(B, C, HW)

    bk2 = bk.reshape(C, 1)
    bq2 = bq.reshape(C, 1)
    bv2 = bv.reshape(C, 1)
    br2 = br.reshape(C, 1)
    alpha2 = alpha.reshape(1, 1)

    wspec = pl.BlockSpec((C, C), lambda b: (0, 0))
    bspec = pl.BlockSpec((C, 1), lambda b: (0, 0))
    xspec = pl.BlockSpec((1, C, HW), lambda b: (b, 0, 0))
    k3, q3, v3 = pl.pallas_call(
        _qkv_kernel,
        grid=(B,),
        in_specs=[xspec, wspec, wspec, wspec, bspec, bspec, bspec],
        out_specs=[xspec, xspec, xspec],
        out_shape=[
            jax.ShapeDtypeStruct((B, C, HW), _BF16),
            jax.ShapeDtypeStruct((B, C, HW), _BF16),
            jax.ShapeDtypeStruct((B, C, HW), _BF16),
        ],
        compiler_params=pltpu.CompilerParams(
            dimension_semantics=("parallel",)),
        name="qkv_proj",
    )(x3, Wk, Wq, Wv, bk2, bq2, bv2)

    BC = _pick_tile(C, 256, 128)
    BD = _pick_tile(C, 256, 128)
    att3 = pl.pallas_call(
        functools.partial(_attn_kernel, B, C // BD, BD),
        grid=(C // BC, C // BD),
        in_specs=[
            pl.BlockSpec((B, BC, HW), lambda c, d: (0, c, 0)),
            pl.BlockSpec(memory_space=pl.ANY),
            pl.BlockSpec((B, BD, HW), lambda c, d: (0, d, 0)),
        ],
        out_specs=pl.BlockSpec((B, BC, HW), lambda c, d: (0, c, 0)),
        out_shape=jax.ShapeDtypeStruct((B, C, HW), _BF16),
        scratch_shapes=[pltpu.VMEM((B, BC, BD), _F32),
                        pltpu.VMEM((B, BC, HW), _BF16),
                        pltpu.VMEM((B, C, HW), _BF16),
                        pltpu.SemaphoreType.DMA],
        compiler_params=pltpu.CompilerParams(
            dimension_semantics=("parallel", "arbitrary"),
            vmem_limit_bytes=62 * 1024 * 1024),
        name="chan_attn",
    )(k3, q3, v3)

    out3 = pl.pallas_call(
        _proj_kernel,
        grid=(B,),
        in_specs=[
            pl.BlockSpec(memory_space=pltpu.SMEM),
            pl.BlockSpec((1, C, HW), lambda b: (b, 0, 0)),
            pl.BlockSpec((1, C, HW), lambda b: (b, 0, 0)),
            pl.BlockSpec((C, C), lambda b: (0, 0)),
            pl.BlockSpec((C, 1), lambda b: (0, 0)),
        ],
        out_specs=pl.BlockSpec((1, C, HW), lambda b: (b, 0, 0)),
        out_shape=jax.ShapeDtypeStruct((B, C, HW), _F32),
        compiler_params=pltpu.CompilerParams(
            dimension_semantics=("parallel",)),
        name="reproj_residual",
    )(alpha2, att3, x3, Wr, br2)

    return out3.reshape(B, C, S0, S1)


# speedup vs baseline: 1.0251x; 1.0251x over previous
import functools

import jax
import jax.numpy as jnp
from jax.experimental import pallas as pl
from jax.experimental.pallas import tpu as pltpu

_F32 = jnp.float32
_BF16 = jnp.bfloat16


def _qkv_kernel(x_ref, wk_ref, wq_ref, wv_ref, bk_ref, bq_ref, bv_ref,
                k_ref, q_ref, v_ref):
    xb = x_ref[0]
    k = jnp.dot(wk_ref[...], xb, preferred_element_type=_F32) + bk_ref[...]
    k_ref[0] = k.astype(_BF16)
    q = jnp.dot(wq_ref[...], xb, preferred_element_type=_F32) + bq_ref[...]
    q_ref[0] = q.astype(_BF16)
    v = jnp.dot(wv_ref[...], xb, preferred_element_type=_F32) + bv_ref[...]
    v_ref[0] = v.astype(_BF16)


def _attn_kernel(nbatch, nd, k_ref, q_ref, v_ref, o_ref, s_ref, e_ref, acc_ref):
    d_idx = pl.program_id(1)

    @pl.when(d_idx == 0)
    def _():
        acc_ref[...] = jnp.zeros_like(acc_ref)

    m = None
    for b in range(nbatch):
        s = jax.lax.dot_general(
            k_ref[b], q_ref[b],
            dimension_numbers=(((1,), (1,)), ((), ())),
            preferred_element_type=_F32)
        s_ref[b] = s
        m = s if m is None else jnp.maximum(m, s)

    norm = jnp.zeros_like(m)
    for b in range(nbatch):
        e = jnp.exp(s_ref[b] - m)
        e_ref[b] = e.astype(_BF16)
        norm = norm + e
    inv_b = (1.0 / norm).astype(_BF16)

    for b in range(nbatch):
        attn = e_ref[b] * inv_b
        pv = jnp.dot(attn, v_ref[b], preferred_element_type=_F32)
        acc_ref[b] += pv

    @pl.when(d_idx == nd - 1)
    def _():
        o_ref[...] = acc_ref[...].astype(o_ref.dtype)


def _proj_kernel(alpha_ref, att_ref, x_ref, wr_ref, br_ref, o_ref):
    a = alpha_ref[0, 0]
    att = jnp.dot(wr_ref[...], att_ref[0].astype(_F32),
                  preferred_element_type=_F32) + br_ref[...]
    o_ref[0] = a * att + x_ref[0]


def _pick_tile(c, target, quantum):
    t = min(target, c)
    while t > quantum and c % t:
        t -= quantum
    return t if c % t == 0 else c


@functools.partial(jax.jit, static_argnames=())
def kernel(x, Wq, bq, Wk, bk, Wv, bv, Wr, br, alpha):
    B, C, S0, S1 = x.shape
    HW = S0 * S1
    x3 = x.reshape(B, C, HW)

    bk2 = bk.reshape(C, 1)
    bq2 = bq.reshape(C, 1)
    bv2 = bv.reshape(C, 1)
    br2 = br.reshape(C, 1)
    alpha2 = alpha.reshape(1, 1)

    wspec = pl.BlockSpec((C, C), lambda b: (0, 0))
    bspec = pl.BlockSpec((C, 1), lambda b: (0, 0))
    xspec = pl.BlockSpec((1, C, HW), lambda b: (b, 0, 0))
    k3, q3, v3 = pl.pallas_call(
        _qkv_kernel,
        grid=(B,),
        in_specs=[xspec, wspec, wspec, wspec, bspec, bspec, bspec],
        out_specs=[xspec, xspec, xspec],
        out_shape=[
            jax.ShapeDtypeStruct((B, C, HW), _BF16),
            jax.ShapeDtypeStruct((B, C, HW), _BF16),
            jax.ShapeDtypeStruct((B, C, HW), _BF16),
        ],
        compiler_params=pltpu.CompilerParams(
            dimension_semantics=("parallel",)),
        name="qkv_proj",
    )(x3, Wk, Wq, Wv, bk2, bq2, bv2)

    BC = _pick_tile(C, 256, 128)
    BD = _pick_tile(C, 256, 128)
    att3 = pl.pallas_call(
        functools.partial(_attn_kernel, B, C // BD),
        grid=(C // BC, C // BD),
        in_specs=[
            pl.BlockSpec((B, BC, HW), lambda c, d: (0, c, 0)),
            pl.BlockSpec((B, BD, HW), lambda c, d: (0, d, 0)),
            pl.BlockSpec((B, BD, HW), lambda c, d: (0, d, 0)),
        ],
        out_specs=pl.BlockSpec((B, BC, HW), lambda c, d: (0, c, 0)),
        out_shape=jax.ShapeDtypeStruct((B, C, HW), _BF16),
        scratch_shapes=[pltpu.VMEM((B, BC, BD), _F32),
                        pltpu.VMEM((B, BC, BD), _BF16),
                        pltpu.VMEM((B, BC, HW), _F32)],
        compiler_params=pltpu.CompilerParams(
            dimension_semantics=("parallel", "arbitrary")),
        name="chan_attn",
    )(k3, q3, v3)

    out3 = pl.pallas_call(
        _proj_kernel,
        grid=(B,),
        in_specs=[
            pl.BlockSpec(memory_space=pltpu.SMEM),
            pl.BlockSpec((1, C, HW), lambda b: (b, 0, 0)),
            pl.BlockSpec((1, C, HW), lambda b: (b, 0, 0)),
            pl.BlockSpec((C, C), lambda b: (0, 0)),
            pl.BlockSpec((C, 1), lambda b: (0, 0)),
        ],
        out_specs=pl.BlockSpec((1, C, HW), lambda b: (b, 0, 0)),
        out_shape=jax.ShapeDtypeStruct((B, C, HW), _F32),
        compiler_params=pltpu.CompilerParams(
            dimension_semantics=("parallel",)),
        name="reproj_residual",
    )(alpha2, att3, x3, Wr, br2)

    return out3.reshape(B, C, S0, S1)


# speedup vs baseline: 1.0371x; 1.0117x over previous
import functools

import jax
import jax.numpy as jnp
from jax.experimental import pallas as pl
from jax.experimental.pallas import tpu as pltpu

_F32 = jnp.float32
_BF16 = jnp.bfloat16


def _qkv_kernel(x_ref, wk_ref, wq_ref, wv_ref, bk_ref, bq_ref, bv_ref,
                k_ref, q_ref, v_ref):
    for b in range(x_ref.shape[0]):
        xb = x_ref[b]
        k = jnp.dot(wk_ref[...], xb, preferred_element_type=_F32) + bk_ref[...]
        k_ref[b] = k.astype(_BF16)
        q = jnp.dot(wq_ref[...], xb, preferred_element_type=_F32) + bq_ref[...]
        q_ref[b] = q.astype(_BF16)
        v = jnp.dot(wv_ref[...], xb, preferred_element_type=_F32) + bv_ref[...]
        v_ref[b] = v.astype(_BF16)


def _attn_kernel(nbatch, nd, k_ref, q_ref, v_ref, o_ref, s_ref, e_ref, acc_ref):
    d_idx = pl.program_id(1)

    @pl.when(d_idx == 0)
    def _():
        acc_ref[...] = jnp.zeros_like(acc_ref)

    m = None
    for b in range(nbatch):
        s = jax.lax.dot_general(
            k_ref[b], q_ref[b],
            dimension_numbers=(((1,), (1,)), ((), ())),
            preferred_element_type=_F32)
        s_ref[b] = s
        m = s if m is None else jnp.maximum(m, s)

    norm = jnp.zeros_like(m)
    for b in range(nbatch):
        e = jnp.exp(s_ref[b] - m)
        e_ref[b] = e.astype(_BF16)
        norm = norm + e
    inv_b = (1.0 / norm).astype(_BF16)

    for b in range(nbatch):
        attn = e_ref[b] * inv_b
        pv = jnp.dot(attn, v_ref[b], preferred_element_type=_F32)
        acc_ref[b] += pv

    @pl.when(d_idx == nd - 1)
    def _():
        o_ref[...] = acc_ref[...].astype(o_ref.dtype)


def _proj_kernel(alpha_ref, att_ref, x_ref, wr_ref, br_ref, o_ref):
    a = alpha_ref[0, 0]
    att = jnp.dot(wr_ref[...], att_ref[0].astype(_F32),
                  preferred_element_type=_F32) + br_ref[...]
    o_ref[0] = a * att + x_ref[0]


def _pick_tile(c, target, quantum):
    t = min(target, c)
    while t > quantum and c % t:
        t -= quantum
    return t if c % t == 0 else c


@functools.partial(jax.jit, static_argnames=())
def kernel(x, Wq, bq, Wk, bk, Wv, bv, Wr, br, alpha):
    B, C, S0, S1 = x.shape
    HW = S0 * S1
    x3 = x.reshape(B, C, HW)

    bk2 = bk.reshape(C, 1)
    bq2 = bq.reshape(C, 1)
    bv2 = bv.reshape(C, 1)
    br2 = br.reshape(C, 1)
    alpha2 = alpha.reshape(1, 1)

    wspec = pl.BlockSpec((C, C), lambda b: (0, 0))
    bspec = pl.BlockSpec((C, 1), lambda b: (0, 0))
    xspec = pl.BlockSpec((2, C, HW), lambda b: (b, 0, 0))
    k3, q3, v3 = pl.pallas_call(
        _qkv_kernel,
        grid=(B // 2,),
        in_specs=[xspec, wspec, wspec, wspec, bspec, bspec, bspec],
        out_specs=[xspec, xspec, xspec],
        out_shape=[
            jax.ShapeDtypeStruct((B, C, HW), _BF16),
            jax.ShapeDtypeStruct((B, C, HW), _BF16),
            jax.ShapeDtypeStruct((B, C, HW), _BF16),
        ],
        compiler_params=pltpu.CompilerParams(
            dimension_semantics=("parallel",)),
        name="qkv_proj",
    )(x3, Wk, Wq, Wv, bk2, bq2, bv2)

    BC = _pick_tile(C, 256, 128)
    BD = _pick_tile(C, 256, 128)
    att3 = pl.pallas_call(
        functools.partial(_attn_kernel, B, C // BD),
        grid=(C // BC, C // BD),
        in_specs=[
            pl.BlockSpec((B, BC, HW), lambda c, d: (0, c, 0)),
            pl.BlockSpec((B, BD, HW), lambda c, d: (0, d, 0)),
            pl.BlockSpec((B, BD, HW), lambda c, d: (0, d, 0)),
        ],
        out_specs=pl.BlockSpec((B, BC, HW), lambda c, d: (0, c, 0)),
        out_shape=jax.ShapeDtypeStruct((B, C, HW), _BF16),
        scratch_shapes=[pltpu.VMEM((B, BC, BD), _F32),
                        pltpu.VMEM((B, BC, BD), _BF16),
                        pltpu.VMEM((B, BC, HW), _F32)],
        compiler_params=pltpu.CompilerParams(
            dimension_semantics=("parallel", "arbitrary")),
        name="chan_attn",
    )(k3, q3, v3)

    out3 = pl.pallas_call(
        _proj_kernel,
        grid=(B,),
        in_specs=[
            pl.BlockSpec(memory_space=pltpu.SMEM),
            pl.BlockSpec((1, C, HW), lambda b: (b, 0, 0)),
            pl.BlockSpec((1, C, HW), lambda b: (b, 0, 0)),
            pl.BlockSpec((C, C), lambda b: (0, 0)),
            pl.BlockSpec((C, 1), lambda b: (0, 0)),
        ],
        out_specs=pl.BlockSpec((1, C, HW), lambda b: (b, 0, 0)),
        out_shape=jax.ShapeDtypeStruct((B, C, HW), _F32),
        compiler_params=pltpu.CompilerParams(
            dimension_semantics=("parallel",)),
        name="reproj_residual",
    )(alpha2, att3, x3, Wr, br2)

    return out3.reshape(B, C, S0, S1)


# speedup vs baseline: 1.0718x; 1.0334x over previous
import functools

import jax
import jax.numpy as jnp
from jax.experimental import pallas as pl
from jax.experimental.pallas import tpu as pltpu

_F32 = jnp.float32
_BF16 = jnp.bfloat16


def _qkv_kernel(x_ref, wk_ref, wq_ref, wv_ref, bk_ref, bq_ref, bv_ref,
                k_ref, q_ref, v_ref):
    for b in range(x_ref.shape[0]):
        xb = x_ref[b]
        k = jnp.dot(wk_ref[...], xb, preferred_element_type=_F32) + bk_ref[...]
        k_ref[b] = k.astype(_BF16)
        q = jnp.dot(wq_ref[...], xb, preferred_element_type=_F32) + bq_ref[...]
        q_ref[b] = q.astype(_BF16)
        v = jnp.dot(wv_ref[...], xb, preferred_element_type=_F32) + bv_ref[...]
        v_ref[b] = v.astype(_BF16)


def _attn_kernel(nbatch, nd, k_ref, q_ref, v_ref, o_ref, s_ref, e_ref, acc_ref):
    d_idx = pl.program_id(1)

    @pl.when(d_idx == 0)
    def _():
        acc_ref[...] = jnp.zeros_like(acc_ref)

    m = None
    for b in range(nbatch):
        s = jax.lax.dot_general(
            k_ref[b], q_ref[b],
            dimension_numbers=(((1,), (1,)), ((), ())),
            preferred_element_type=_F32)
        s_ref[b] = s
        m = s if m is None else jnp.maximum(m, s)

    norm = jnp.zeros_like(m)
    for b in range(nbatch):
        e = jnp.exp(s_ref[b] - m)
        e_ref[b] = e.astype(_BF16)
        norm = norm + e
    inv_b = (1.0 / norm).astype(_BF16)

    for b in range(nbatch):
        attn = e_ref[b] * inv_b
        pv = jnp.dot(attn, v_ref[b], preferred_element_type=_F32)
        acc_ref[b] += pv

    @pl.when(d_idx == nd - 1)
    def _():
        o_ref[...] = acc_ref[...].astype(o_ref.dtype)


def _proj_kernel(alpha_ref, att_ref, x_ref, wr_ref, br_ref, o_ref):
    a = alpha_ref[0, 0]
    for b in range(x_ref.shape[0]):
        att = jnp.dot(wr_ref[...], att_ref[b].astype(_F32),
                      preferred_element_type=_F32) + br_ref[...]
        o_ref[b] = a * att + x_ref[b]


def _pick_tile(c, target, quantum):
    t = min(target, c)
    while t > quantum and c % t:
        t -= quantum
    return t if c % t == 0 else c


@functools.partial(jax.jit, static_argnames=())
def kernel(x, Wq, bq, Wk, bk, Wv, bv, Wr, br, alpha):
    B, C, S0, S1 = x.shape
    HW = S0 * S1
    x3 = x.reshape(B, C, HW)

    bk2 = bk.reshape(C, 1)
    bq2 = bq.reshape(C, 1)
    bv2 = bv.reshape(C, 1)
    br2 = br.reshape(C, 1)
    alpha2 = alpha.reshape(1, 1)

    wspec = pl.BlockSpec((C, C), lambda b: (0, 0))
    bspec = pl.BlockSpec((C, 1), lambda b: (0, 0))
    xspec = pl.BlockSpec((2, C, HW), lambda b: (b, 0, 0))
    k3, q3, v3 = pl.pallas_call(
        _qkv_kernel,
        grid=(B // 2,),
        in_specs=[xspec, wspec, wspec, wspec, bspec, bspec, bspec],
        out_specs=[xspec, xspec, xspec],
        out_shape=[
            jax.ShapeDtypeStruct((B, C, HW), _BF16),
            jax.ShapeDtypeStruct((B, C, HW), _BF16),
            jax.ShapeDtypeStruct((B, C, HW), _BF16),
        ],
        compiler_params=pltpu.CompilerParams(
            dimension_semantics=("parallel",)),
        name="qkv_proj",
    )(x3, Wk, Wq, Wv, bk2, bq2, bv2)

    BC = _pick_tile(C, 256, 128)
    BD = _pick_tile(C, 256, 128)
    att3 = pl.pallas_call(
        functools.partial(_attn_kernel, B, C // BD),
        grid=(C // BC, C // BD),
        in_specs=[
            pl.BlockSpec((B, BC, HW), lambda c, d: (0, c, 0)),
            pl.BlockSpec((B, BD, HW), lambda c, d: (0, d, 0)),
            pl.BlockSpec((B, BD, HW), lambda c, d: (0, d, 0)),
        ],
        out_specs=pl.BlockSpec((B, BC, HW), lambda c, d: (0, c, 0)),
        out_shape=jax.ShapeDtypeStruct((B, C, HW), _BF16),
        scratch_shapes=[pltpu.VMEM((B, BC, BD), _F32),
                        pltpu.VMEM((B, BC, BD), _BF16),
                        pltpu.VMEM((B, BC, HW), _F32)],
        compiler_params=pltpu.CompilerParams(
            dimension_semantics=("parallel", "arbitrary")),
        name="chan_attn",
    )(k3, q3, v3)

    pspec = pl.BlockSpec((2, C, HW), lambda b: (b, 0, 0))
    out3 = pl.pallas_call(
        _proj_kernel,
        grid=(B // 2,),
        in_specs=[
            pl.BlockSpec(memory_space=pltpu.SMEM),
            pspec,
            pspec,
            pl.BlockSpec((C, C), lambda b: (0, 0)),
            pl.BlockSpec((C, 1), lambda b: (0, 0)),
        ],
        out_specs=pspec,
        out_shape=jax.ShapeDtypeStruct((B, C, HW), _F32),
        compiler_params=pltpu.CompilerParams(
            dimension_semantics=("parallel",)),
        name="reproj_residual",
    )(alpha2, att3, x3, Wr, br2)

    return out3.reshape(B, C, S0, S1)


# speedup vs baseline: 1.0838x; 1.0112x over previous
import functools

import jax
import jax.numpy as jnp
from jax.experimental import pallas as pl
from jax.experimental.pallas import tpu as pltpu

_F32 = jnp.float32
_BF16 = jnp.bfloat16


def _qkv_kernel(x_ref, wk_ref, wq_ref, wv_ref, bk_ref, bq_ref, bv_ref,
                k_ref, q_ref, v_ref):
    for b in range(x_ref.shape[0]):
        xb = x_ref[b]
        k = jnp.dot(wk_ref[...], xb, preferred_element_type=_F32) + bk_ref[...]
        k_ref[b] = k.astype(_BF16)
        q = jnp.dot(wq_ref[...], xb, preferred_element_type=_F32) + bq_ref[...]
        q_ref[b] = q.astype(_BF16)
        v = jnp.dot(wv_ref[...], xb, preferred_element_type=_F32) + bv_ref[...]
        v_ref[b] = v.astype(_BF16)


def _attn_kernel(nbatch, nd, k_ref, q_ref, v_ref, o_ref, s_ref, e_ref, acc_ref):
    d_idx = pl.program_id(1)

    @pl.when(d_idx == 0)
    def _():
        acc_ref[...] = jnp.zeros_like(acc_ref)

    m = None
    for b in range(nbatch):
        s = jax.lax.dot_general(
            k_ref[b], q_ref[b],
            dimension_numbers=(((1,), (1,)), ((), ())),
            preferred_element_type=_F32)
        s_ref[b] = s
        m = s if m is None else jnp.maximum(m, s)

    norm = jnp.zeros_like(m)
    for b in range(nbatch):
        e = jnp.exp(s_ref[b] - m)
        e_ref[b] = e.astype(_BF16)
        norm = norm + e
    inv_b = (1.0 / norm).astype(_BF16)

    for b in range(nbatch):
        attn = e_ref[b] * inv_b
        pv = jnp.dot(attn, v_ref[b], preferred_element_type=_F32)
        acc_ref[b] += pv

    @pl.when(d_idx == nd - 1)
    def _():
        o_ref[...] = acc_ref[...].astype(o_ref.dtype)


def _proj_kernel(alpha_ref, att_ref, x_ref, wr_ref, br_ref, o_ref):
    a = alpha_ref[0, 0]
    for b in range(x_ref.shape[0]):
        att = jnp.dot(wr_ref[...], att_ref[b].astype(_F32),
                      preferred_element_type=_F32) + br_ref[...]
        o_ref[b] = a * att + x_ref[b]


def _pick_tile(c, target, quantum):
    t = min(target, c)
    while t > quantum and c % t:
        t -= quantum
    return t if c % t == 0 else c


@functools.partial(jax.jit, static_argnames=())
def kernel(x, Wq, bq, Wk, bk, Wv, bv, Wr, br, alpha):
    B, C, S0, S1 = x.shape
    HW = S0 * S1
    x3 = x.reshape(B, C, HW)

    bk2 = bk.reshape(C, 1)
    bq2 = bq.reshape(C, 1)
    bv2 = bv.reshape(C, 1)
    br2 = br.reshape(C, 1)
    alpha2 = alpha.reshape(1, 1)

    wspec = pl.BlockSpec((C, C), lambda b: (0, 0))
    bspec = pl.BlockSpec((C, 1), lambda b: (0, 0))
    xspec = pl.BlockSpec((2, C, HW), lambda b: (b, 0, 0))
    k3, q3, v3 = pl.pallas_call(
        _qkv_kernel,
        grid=(B // 2,),
        in_specs=[xspec, wspec, wspec, wspec, bspec, bspec, bspec],
        out_specs=[xspec, xspec, xspec],
        out_shape=[
            jax.ShapeDtypeStruct((B, C, HW), _BF16),
            jax.ShapeDtypeStruct((B, C, HW), _BF16),
            jax.ShapeDtypeStruct((B, C, HW), _BF16),
        ],
        compiler_params=pltpu.CompilerParams(
            dimension_semantics=("parallel",)),
        name="qkv_proj",
    )(x3, Wk, Wq, Wv, bk2, bq2, bv2)

    BC = _pick_tile(C, 256, 128)
    BD = _pick_tile(C, 256, 128)
    att3 = pl.pallas_call(
        functools.partial(_attn_kernel, B, C // BD),
        grid=(C // BC, C // BD),
        in_specs=[
            pl.BlockSpec((B, BC, HW), lambda c, d: (0, c, 0)),
            pl.BlockSpec((B, BD, HW), lambda c, d: (0, d, 0)),
            pl.BlockSpec((B, BD, HW), lambda c, d: (0, d, 0)),
        ],
        out_specs=pl.BlockSpec((B, BC, HW), lambda c, d: (0, c, 0)),
        out_shape=jax.ShapeDtypeStruct((B, C, HW), _BF16),
        scratch_shapes=[pltpu.VMEM((B, BC, BD), _F32),
                        pltpu.VMEM((B, BC, BD), _BF16),
                        pltpu.VMEM((B, BC, HW), _F32)],
        compiler_params=pltpu.CompilerParams(
            dimension_semantics=("parallel", "arbitrary")),
        name="chan_attn",
    )(k3, q3, v3)

    pspec = pl.BlockSpec((4, C, HW), lambda b: (b, 0, 0))
    out3 = pl.pallas_call(
        _proj_kernel,
        grid=(B // 4,),
        in_specs=[
            pl.BlockSpec(memory_space=pltpu.SMEM),
            pspec,
            pspec,
            pl.BlockSpec((C, C), lambda b: (0, 0)),
            pl.BlockSpec((C, 1), lambda b: (0, 0)),
        ],
        out_specs=pspec,
        out_shape=jax.ShapeDtypeStruct((B, C, HW), _F32),
        compiler_params=pltpu.CompilerParams(
            dimension_semantics=("parallel",)),
        name="reproj_residual",
    )(alpha2, att3, x3, Wr, br2)

    return out3.reshape(B, C, S0, S1)


# speedup vs baseline: 1.0862x; 1.0023x over previous
import functools

import jax
import jax.numpy as jnp
from jax.experimental import pallas as pl
from jax.experimental.pallas import tpu as pltpu

_F32 = jnp.float32
_BF16 = jnp.bfloat16


def _qkv_kernel(x_ref, wk_ref, wq_ref, wv_ref, bk_ref, bq_ref, bv_ref,
                k_ref, q_ref, v_ref):
    for b in range(x_ref.shape[0]):
        xb = x_ref[b]
        k = jnp.dot(wk_ref[...], xb, preferred_element_type=_F32) + bk_ref[...]
        k_ref[b] = k.astype(_BF16)
        q = jnp.dot(wq_ref[...], xb, preferred_element_type=_F32) + bq_ref[...]
        q_ref[b] = q.astype(_BF16)
        v = jnp.dot(wv_ref[...], xb, preferred_element_type=_F32) + bv_ref[...]
        v_ref[b] = v.astype(_BF16)


def _attn_kernel(nbatch, nd, k_ref, q_ref, v_ref, o_ref, s_ref, e_ref, acc_ref):
    d_idx = pl.program_id(1)

    @pl.when(d_idx == 0)
    def _():
        acc_ref[...] = jnp.zeros_like(acc_ref)

    m = None
    for b in range(nbatch):
        s = jax.lax.dot_general(
            k_ref[b], q_ref[b],
            dimension_numbers=(((1,), (1,)), ((), ())),
            preferred_element_type=_F32)
        s_ref[b] = s
        m = s if m is None else jnp.maximum(m, s)

    norm = jnp.zeros_like(m)
    for b in range(nbatch):
        e = jnp.exp(s_ref[b] - m)
        e_ref[b] = e.astype(_BF16)
        norm = norm + e
    inv_b = (1.0 / norm).astype(_BF16)

    for b in range(nbatch):
        attn = e_ref[b] * inv_b
        pv = jnp.dot(attn, v_ref[b], preferred_element_type=_F32)
        acc_ref[b] += pv

    @pl.when(d_idx == nd - 1)
    def _():
        o_ref[...] = acc_ref[...].astype(o_ref.dtype)


def _proj_kernel(alpha_ref, att_ref, x_ref, wr_ref, br_ref, o_ref):
    a = alpha_ref[0, 0]
    for b in range(x_ref.shape[0]):
        att = jnp.dot(wr_ref[...], att_ref[b].astype(_F32),
                      preferred_element_type=_F32) + br_ref[...]
        o_ref[b] = a * att + x_ref[b]


def _pick_tile(c, target, quantum):
    t = min(target, c)
    while t > quantum and c % t:
        t -= quantum
    return t if c % t == 0 else c


@functools.partial(jax.jit, static_argnames=())
def kernel(x, Wq, bq, Wk, bk, Wv, bv, Wr, br, alpha):
    B, C, S0, S1 = x.shape
    HW = S0 * S1
    x3 = x.reshape(B, C, HW)

    bk2 = bk.reshape(C, 1)
    bq2 = bq.reshape(C, 1)
    bv2 = bv.reshape(C, 1)
    br2 = br.reshape(C, 1)
    alpha2 = alpha.reshape(1, 1)

    wspec = pl.BlockSpec((C, C), lambda b: (0, 0))
    bspec = pl.BlockSpec((C, 1), lambda b: (0, 0))
    xspec = pl.BlockSpec((4, C, HW), lambda b: (b, 0, 0))
    k3, q3, v3 = pl.pallas_call(
        _qkv_kernel,
        grid=(B // 4,),
        in_specs=[xspec, wspec, wspec, wspec, bspec, bspec, bspec],
        out_specs=[xspec, xspec, xspec],
        out_shape=[
            jax.ShapeDtypeStruct((B, C, HW), _BF16),
            jax.ShapeDtypeStruct((B, C, HW), _BF16),
            jax.ShapeDtypeStruct((B, C, HW), _BF16),
        ],
        compiler_params=pltpu.CompilerParams(
            dimension_semantics=("parallel",),
            vmem_limit_bytes=63 * 1024 * 1024),
        name="qkv_proj",
    )(x3, Wk, Wq, Wv, bk2, bq2, bv2)

    BC = _pick_tile(C, 256, 128)
    BD = _pick_tile(C, 256, 128)
    att3 = pl.pallas_call(
        functools.partial(_attn_kernel, B, C // BD),
        grid=(C // BC, C // BD),
        in_specs=[
            pl.BlockSpec((B, BC, HW), lambda c, d: (0, c, 0)),
            pl.BlockSpec((B, BD, HW), lambda c, d: (0, d, 0)),
            pl.BlockSpec((B, BD, HW), lambda c, d: (0, d, 0)),
        ],
        out_specs=pl.BlockSpec((B, BC, HW), lambda c, d: (0, c, 0)),
        out_shape=jax.ShapeDtypeStruct((B, C, HW), _BF16),
        scratch_shapes=[pltpu.VMEM((B, BC, BD), _F32),
                        pltpu.VMEM((B, BC, BD), _BF16),
                        pltpu.VMEM((B, BC, HW), _F32)],
        compiler_params=pltpu.CompilerParams(
            dimension_semantics=("parallel", "arbitrary")),
        name="chan_attn",
    )(k3, q3, v3)

    pspec = pl.BlockSpec((4, C, HW), lambda b: (b, 0, 0))
    out3 = pl.pallas_call(
        _proj_kernel,
        grid=(B // 4,),
        in_specs=[
            pl.BlockSpec(memory_space=pltpu.SMEM),
            pspec,
            pspec,
            pl.BlockSpec((C, C), lambda b: (0, 0)),
            pl.BlockSpec((C, 1), lambda b: (0, 0)),
        ],
        out_specs=pspec,
        out_shape=jax.ShapeDtypeStruct((B, C, HW), _F32),
        compiler_params=pltpu.CompilerParams(
            dimension_semantics=("parallel",)),
        name="reproj_residual",
    )(alpha2, att3, x3, Wr, br2)

    return out3.reshape(B, C, S0, S1)
